# Initial kernel scaffold; baseline (speedup 1.0000x reference)
import jax
import jax.numpy as jnp
from jax.experimental import pallas as pl


def kernel(hidden_states, Wq, Wk, Wv, Wo):
    raise NotImplementedError("write your pallas kernel here")


# speedup vs baseline: 2.5557x; 2.5557x over previous
import functools

import jax
import jax.numpy as jnp
from jax import lax
from jax.experimental import pallas as pl
from jax.experimental.pallas import tpu as pltpu
from jax.experimental.pallas import tpu_sc as plsc

HIDDEN = 768
NUM_HEADS = 12
HEAD_DIM = 64
BLOCK = 512
SEQ = 2048
NB = SEQ // BLOCK
SCALE = 0.125
NEG = -1e30
LN2 = 0.6931471805599453

_SC_CORES = 2
_SC_SUBCORES = 16
_NW = _SC_CORES * _SC_SUBCORES
_CHUNK = SEQ // _NW
_NROW = NUM_HEADS * NB


def _qkv_body(x_ref, w_ref, cos_ref, sin_ref, y_ref, ksum_ref):
    x = x_ref[...]
    w = w_ref[...]
    y = jnp.dot(x, w, preferred_element_type=jnp.float32)
    cos = cos_ref[...]
    sin = sin_ref[...]
    cos_t = jnp.concatenate([cos] * NUM_HEADS, axis=1)
    sin_t = jnp.concatenate([sin] * NUM_HEADS, axis=1)
    q = y[:, :HIDDEN]
    k = y[:, HIDDEN:2 * HIDDEN]
    v = y[:, 2 * HIDDEN:]

    def rot(z):
        parts = []
        for h in range(NUM_HEADS):
            zh = z[:, h * HEAD_DIM:(h + 1) * HEAD_DIM]
            parts.append(
                jnp.concatenate([-zh[:, HEAD_DIM // 2:], zh[:, :HEAD_DIM // 2]],
                                axis=1))
        return jnp.concatenate(parts, axis=1)

    q = q * cos_t + rot(q) * sin_t
    k = k * cos_t + rot(k) * sin_t
    y_ref[...] = jnp.concatenate([q, k, v], axis=1)
    ksum_ref[...] = jnp.sum(k, axis=0)[None, None, :]


def _aff_body(q_ref, ks_ref, aff_ref):
    ks = ks_ref[:, 0, :]
    for h in range(NUM_HEADS):
        kh = ks[:, h * HEAD_DIM:(h + 1) * HEAD_DIM]
        qh = q_ref[:, h * HEAD_DIM:(h + 1) * HEAD_DIM]
        a = lax.dot_general(kh, qh, (((1,), (1,)), ((), ())),
                            preferred_element_type=jnp.float32)
        aff_ref[h * NB:(h + 1) * NB, :] = a * (SCALE / BLOCK)


def _route_body(aff_hbm, bias_hbm, abuf, bbuf):
    wid = lax.axis_index("s") * _SC_CORES + lax.axis_index("c")
    pltpu.sync_copy(aff_hbm.at[wid], abuf)

    def b2i(c):
        return jnp.where(c, jnp.int32(1), jnp.int32(0))

    def load_masked(h, g, cur):
        a = []
        for j in range(NB):
            aj = abuf[h * NB + j, pl.ds(g * 16, 16)]
            a.append(jnp.where(jnp.int32(j) > cur, NEG, aj))
        return a

    def ranks(a):
        r = []
        for j in range(NB):
            rj = None
            for jp in range(NB):
                if jp == j:
                    continue
                c = a[jp] > a[j]
                if jp < j:
                    c = c | (a[jp] == a[j])
                rj = b2i(c) if rj is None else rj + b2i(c)
            r.append(rj)
        return r

    for g in range(_CHUNK // 16):
        ids = lax.broadcasted_iota(jnp.int32, (16,), 0)
        pos = ids + (wid * _CHUNK + g * 16)
        cur = lax.shift_right_logical(pos, 9)

        allhave = pos >= 0
        for h in range(NUM_HEADS):
            a = load_masked(h, g, cur)
            acur = a[NB - 1]
            for j in range(NB - 2, -1, -1):
                acur = jnp.where(cur == j, a[j], acur)
            rc = None
            for j in range(NB):
                c = (a[j] > acur) | ((a[j] == acur) & (jnp.int32(j) < cur))
                rc = b2i(c) if rc is None else rc + b2i(c)
            allhave = allhave & (rc < 2)

        for h in range(NUM_HEADS):
            a = load_masked(h, g, cur)
            r = ranks(a)
            v0 = jnp.maximum(jnp.maximum(a[0], a[1]), jnp.maximum(a[2], a[3]))
            v1 = jnp.where(r[0] == 1, a[0], NEG)
            for j in range(1, NB):
                v1 = jnp.where(r[j] == 1, a[j], v1)
            tie = v0 == v1
            kept_rank = b2i(tie)
            kept = None
            for j in range(NB):
                kj = jnp.where(r[j] == kept_rank, jnp.int32(j), jnp.int32(0))
                kept = kj if kept is None else kept + kj
            for j in range(NB):
                sel = b2i(r[j] < 2)
                fixed = b2i(kept == j) + b2i(cur == j)
                cnt = jnp.where(allhave, sel, fixed)
                bias = jnp.where(cnt == 0, jnp.float32(NEG),
                                 jnp.where(cnt == 2, jnp.float32(LN2),
                                           jnp.float32(0.0)))
                bbuf[h * NB + j, pl.ds(g * 16, 16)] = bias

    pltpu.sync_copy(bbuf, bias_hbm.at[wid])


@functools.cache
def _get_route():
    return pl.kernel(
        _route_body,
        out_type=jax.ShapeDtypeStruct((_NW, _NROW, _CHUNK), jnp.float32),
        mesh=plsc.VectorSubcoreMesh(
            core_axis_name="c", subcore_axis_name="s",
            num_cores=_SC_CORES, num_subcores=_SC_SUBCORES),
        scratch_types=[
            pltpu.VMEM((_NROW, _CHUNK), jnp.float32),
            pltpu.VMEM((_NROW, _CHUNK), jnp.float32),
        ],
    )


def _attn_body(q_ref, k_ref, v_ref, b_ref, wo_ref, o_ref):
    h = pl.program_id(0)
    q = q_ref[0]
    k = k_ref[0]
    v = v_ref[0]
    b = b_ref[0]
    wo = wo_ref[0]
    outs = []
    for qb in range(NB):
        nkb = (2, 2, 3, 4)[qb]
        nk = nkb * BLOCK
        qs = q[qb * BLOCK:(qb + 1) * BLOCK]
        s = lax.dot_general(qs, k[:nk], (((1,), (1,)), ((), ())),
                            preferred_element_type=jnp.float32) * SCALE
        bias = jnp.concatenate(
            [jnp.broadcast_to(b[qb * BLOCK:(qb + 1) * BLOCK, j:j + 1],
                              (BLOCK, BLOCK)) for j in range(nkb)], axis=1)
        s = s + bias
        m = jnp.max(s, axis=1, keepdims=True)
        e = jnp.exp(s - m)
        den = jnp.sum(e, axis=1, keepdims=True)
        o = lax.dot_general(e, v[:nk], (((1,), (0,)), ((), ())),
                            preferred_element_type=jnp.float32) / den
        outs.append(o)
    attn = jnp.concatenate(outs, axis=0)
    contrib = jnp.dot(attn, wo, preferred_element_type=jnp.float32)

    @pl.when(h == 0)
    def _():
        o_ref[...] = contrib

    @pl.when(h > 0)
    def _():
        o_ref[...] = o_ref[...] + contrib


@jax.jit
def kernel(hidden_states, Wq, Wk, Wv, Wo):
    x = hidden_states[0]
    w_qkv = jnp.concatenate([Wq, Wk, Wv], axis=0).T

    inv = 1.0 / (10000.0 ** (jnp.arange(0, HEAD_DIM, 2, dtype=jnp.float32)
                             / HEAD_DIM))
    t = jnp.arange(SEQ, dtype=jnp.float32)
    freqs = t[:, None] * inv[None, :]
    cos64 = jnp.concatenate([jnp.cos(freqs)] * 2, axis=1)
    sin64 = jnp.concatenate([jnp.sin(freqs)] * 2, axis=1)

    y, ksum = pl.pallas_call(
        _qkv_body,
        grid=(NB,),
        in_specs=[
            pl.BlockSpec((BLOCK, HIDDEN), lambda i: (i, 0)),
            pl.BlockSpec((HIDDEN, 3 * HIDDEN), lambda i: (0, 0)),
            pl.BlockSpec((BLOCK, HEAD_DIM), lambda i: (i, 0)),
            pl.BlockSpec((BLOCK, HEAD_DIM), lambda i: (i, 0)),
        ],
        out_specs=[
            pl.BlockSpec((BLOCK, 3 * HIDDEN), lambda i: (i, 0)),
            pl.BlockSpec((1, 1, HIDDEN), lambda i: (i, 0, 0)),
        ],
        out_shape=[
            jax.ShapeDtypeStruct((SEQ, 3 * HIDDEN), jnp.float32),
            jax.ShapeDtypeStruct((NB, 1, HIDDEN), jnp.float32),
        ],
    )(x, w_qkv, cos64, sin64)

    aff = pl.pallas_call(
        _aff_body,
        grid=(1,),
        in_specs=[
            pl.BlockSpec((SEQ, HIDDEN), lambda i: (0, 0)),
            pl.BlockSpec((NB, 1, HIDDEN), lambda i: (0, 0, 0)),
        ],
        out_specs=pl.BlockSpec((_NROW, SEQ), lambda i: (0, 0)),
        out_shape=jax.ShapeDtypeStruct((_NROW, SEQ), jnp.float32),
    )(y, ksum)

    aff_sc = aff.reshape(_NROW, _NW, _CHUNK).transpose(1, 0, 2)
    bias_sc = _get_route()(aff_sc)
    bias = (bias_sc.reshape(_NW, NUM_HEADS, NB, _CHUNK)
            .transpose(1, 0, 3, 2).reshape(NUM_HEADS, SEQ, NB))

    q_all = y[:, :HIDDEN].reshape(SEQ, NUM_HEADS, HEAD_DIM).transpose(1, 0, 2)
    k_all = y[:, HIDDEN:2 * HIDDEN].reshape(SEQ, NUM_HEADS,
                                            HEAD_DIM).transpose(1, 0, 2)
    v_all = y[:, 2 * HIDDEN:].reshape(SEQ, NUM_HEADS, HEAD_DIM).transpose(1, 0, 2)
    wo_r = Wo.T.reshape(NUM_HEADS, HEAD_DIM, HIDDEN)

    out = pl.pallas_call(
        _attn_body,
        grid=(NUM_HEADS,),
        in_specs=[
            pl.BlockSpec((1, SEQ, HEAD_DIM), lambda h: (h, 0, 0)),
            pl.BlockSpec((1, SEQ, HEAD_DIM), lambda h: (h, 0, 0)),
            pl.BlockSpec((1, SEQ, HEAD_DIM), lambda h: (h, 0, 0)),
            pl.BlockSpec((1, SEQ, NB), lambda h: (h, 0, 0)),
            pl.BlockSpec((1, HEAD_DIM, HIDDEN), lambda h: (h, 0, 0)),
        ],
        out_specs=pl.BlockSpec((SEQ, HIDDEN), lambda h: (0, 0)),
        out_shape=jax.ShapeDtypeStruct((SEQ, HIDDEN), jnp.float32),
    )(q_all, k_all, v_all, bias, wo_r)

    return out[None]


# speedup vs baseline: 3.2298x; 1.2638x over previous
import functools

import jax
import jax.numpy as jnp
from jax import lax
from jax.experimental import pallas as pl
from jax.experimental.pallas import tpu as pltpu
from jax.experimental.pallas import tpu_sc as plsc

HIDDEN = 768
NUM_HEADS = 12
HEAD_DIM = 64
BLOCK = 512
SEQ = 2048
NB = SEQ // BLOCK
SCALE = 0.125
NEG = -1e30
LN2 = 0.6931471805599453

_SC_CORES = 2
_SC_SUBCORES = 16
_NW = _SC_CORES * _SC_SUBCORES
_CHUNK = SEQ // _NW
_NROW = NUM_HEADS * NB


def _qkv_body(x_ref, w_ref, cos_ref, sin_ref, q_ref, k_ref, v_ref, ksum_ref):
    x = x_ref[...]
    w = w_ref[...]
    y = jnp.dot(x, w, preferred_element_type=jnp.float32)
    cos = cos_ref[...]
    sin = sin_ref[...]
    cos_t = jnp.concatenate([cos] * NUM_HEADS, axis=1)
    sin_t = jnp.concatenate([sin] * NUM_HEADS, axis=1)
    q = y[:, :HIDDEN]
    k = y[:, HIDDEN:2 * HIDDEN]

    def rot(z):
        parts = []
        for h in range(NUM_HEADS):
            zh = z[:, h * HEAD_DIM:(h + 1) * HEAD_DIM]
            parts.append(
                jnp.concatenate([-zh[:, HEAD_DIM // 2:], zh[:, :HEAD_DIM // 2]],
                                axis=1))
        return jnp.concatenate(parts, axis=1)

    q = q * cos_t + rot(q) * sin_t
    k = k * cos_t + rot(k) * sin_t
    for h in range(NUM_HEADS):
        q_ref[h] = q[:, h * HEAD_DIM:(h + 1) * HEAD_DIM]
        k_ref[h] = k[:, h * HEAD_DIM:(h + 1) * HEAD_DIM]
        v_ref[h] = y[:, 2 * HIDDEN + h * HEAD_DIM:2 * HIDDEN + (h + 1) * HEAD_DIM]
    ksum_ref[...] = jnp.sum(k, axis=0)[None, None, :]


def _aff_body(q_ref, ks_ref, aff_ref):
    ks = ks_ref[:, 0, :]
    for h in range(NUM_HEADS):
        kh = ks[:, h * HEAD_DIM:(h + 1) * HEAD_DIM]
        qh = q_ref[h]
        a = lax.dot_general(kh, qh, (((1,), (1,)), ((), ())),
                            preferred_element_type=jnp.float32)
        a = a * (SCALE / BLOCK)
        for w in range(_NW):
            aff_ref[w, h * NB:(h + 1) * NB, :] = a[:, w * _CHUNK:(w + 1) * _CHUNK]


def _route_body(aff_hbm, bias_hbm, abuf, bbuf):
    wid = lax.axis_index("s") * _SC_CORES + lax.axis_index("c")
    base = wid * _CHUNK
    pltpu.sync_copy(aff_hbm.at[wid], abuf)

    def b2i(c):
        return jnp.where(c, jnp.int32(1), jnp.int32(0))

    def load_masked(h, g, cur):
        a = []
        for j in range(NB):
            aj = abuf[h * NB + j, pl.ds(g * 16, 16)]
            a.append(jnp.where(jnp.int32(j) > cur, NEG, aj))
        return a

    def ranks(a):
        r = []
        for j in range(NB):
            rj = None
            for jp in range(NB):
                if jp == j:
                    continue
                c = a[jp] > a[j]
                if jp < j:
                    c = c | (a[jp] == a[j])
                rj = b2i(c) if rj is None else rj + b2i(c)
            r.append(rj)
        return r

    for g in range(_CHUNK // 16):
        ids = lax.broadcasted_iota(jnp.int32, (16,), 0)
        pos = ids + (base + g * 16)
        cur = lax.shift_right_logical(pos, 9)

        allhave = pos >= 0
        for h in range(NUM_HEADS):
            a = load_masked(h, g, cur)
            acur = a[NB - 1]
            for j in range(NB - 2, -1, -1):
                acur = jnp.where(cur == j, a[j], acur)
            rc = None
            for j in range(NB):
                c = (a[j] > acur) | ((a[j] == acur) & (jnp.int32(j) < cur))
                rc = b2i(c) if rc is None else rc + b2i(c)
            allhave = allhave & (rc < 2)

        for h in range(NUM_HEADS):
            a = load_masked(h, g, cur)
            r = ranks(a)
            v0 = jnp.maximum(jnp.maximum(a[0], a[1]), jnp.maximum(a[2], a[3]))
            v1 = jnp.where(r[0] == 1, a[0], NEG)
            for j in range(1, NB):
                v1 = jnp.where(r[j] == 1, a[j], v1)
            tie = v0 == v1
            kept_rank = b2i(tie)
            kept = None
            for j in range(NB):
                kj = jnp.where(r[j] == kept_rank, jnp.int32(j), jnp.int32(0))
                kept = kj if kept is None else kept + kj
            for j in range(NB):
                sel = b2i(r[j] < 2)
                fixed = b2i(kept == j) + b2i(cur == j)
                cnt = jnp.where(allhave, sel, fixed)
                bias = jnp.where(cnt == 0, jnp.float32(NEG),
                                 jnp.where(cnt == 2, jnp.float32(LN2),
                                           jnp.float32(0.0)))
                bbuf[h * NB + j, pl.ds(g * 16, 16)] = bias

    pltpu.sync_copy(bbuf, bias_hbm.at[wid])


@functools.cache
def _get_route():
    return pl.kernel(
        _route_body,
        out_type=jax.ShapeDtypeStruct((_NW, _NROW, _CHUNK), jnp.float32),
        mesh=plsc.VectorSubcoreMesh(
            core_axis_name="c", subcore_axis_name="s",
            num_cores=_SC_CORES, num_subcores=_SC_SUBCORES),
        scratch_types=[
            pltpu.VMEM((_NROW, _CHUNK), jnp.float32),
            pltpu.VMEM((_NROW, _CHUNK), jnp.float32),
        ],
    )


def _attn_body(q_ref, k_ref, v_ref, b_ref, wo_ref, o_ref, acc_ref):
    h = pl.program_id(0)
    q = q_ref[0]
    k = k_ref[0]
    v = v_ref[0]
    b = b_ref[0]
    outs = []
    for qb in range(NB):
        nkb = (2, 2, 3, 4)[qb]
        nk = nkb * BLOCK
        qs = q[qb * BLOCK:(qb + 1) * BLOCK]
        s = lax.dot_general(qs, k[:nk], (((1,), (1,)), ((), ())),
                            preferred_element_type=jnp.float32) * SCALE
        bias = jnp.concatenate(
            [jnp.broadcast_to(b[qb * BLOCK:(qb + 1) * BLOCK, j:j + 1],
                              (BLOCK, BLOCK)) for j in range(nkb)], axis=1)
        s = s + bias
        m = jnp.max(s, axis=1, keepdims=True)
        e = jnp.exp(s - m)
        den = jnp.sum(e, axis=1, keepdims=True)
        o = lax.dot_general(e, v[:nk], (((1,), (0,)), ((), ())),
                            preferred_element_type=jnp.float32) / den
        outs.append(o)
    acc_ref[h] = jnp.concatenate(outs, axis=0)

    @pl.when(h == NUM_HEADS - 1)
    def _():
        acc = None
        for hh in range(NUM_HEADS):
            c = jnp.dot(acc_ref[hh], wo_ref[hh],
                        preferred_element_type=jnp.float32)
            acc = c if acc is None else acc + c
        o_ref[...] = acc


@jax.jit
def kernel(hidden_states, Wq, Wk, Wv, Wo):
    x = hidden_states[0]
    w_qkv = jnp.concatenate([Wq, Wk, Wv], axis=0).T

    inv = 1.0 / (10000.0 ** (jnp.arange(0, HEAD_DIM, 2, dtype=jnp.float32)
                             / HEAD_DIM))
    t = jnp.arange(SEQ, dtype=jnp.float32)
    freqs = t[:, None] * inv[None, :]
    cos64 = jnp.concatenate([jnp.cos(freqs)] * 2, axis=1)
    sin64 = jnp.concatenate([jnp.sin(freqs)] * 2, axis=1)

    q_all, k_all, v_all, ksum = pl.pallas_call(
        _qkv_body,
        grid=(NB,),
        in_specs=[
            pl.BlockSpec((BLOCK, HIDDEN), lambda i: (i, 0)),
            pl.BlockSpec((HIDDEN, 3 * HIDDEN), lambda i: (0, 0)),
            pl.BlockSpec((BLOCK, HEAD_DIM), lambda i: (i, 0)),
            pl.BlockSpec((BLOCK, HEAD_DIM), lambda i: (i, 0)),
        ],
        out_specs=[
            pl.BlockSpec((NUM_HEADS, BLOCK, HEAD_DIM), lambda i: (0, i, 0)),
            pl.BlockSpec((NUM_HEADS, BLOCK, HEAD_DIM), lambda i: (0, i, 0)),
            pl.BlockSpec((NUM_HEADS, BLOCK, HEAD_DIM), lambda i: (0, i, 0)),
            pl.BlockSpec((1, 1, HIDDEN), lambda i: (i, 0, 0)),
        ],
        out_shape=[
            jax.ShapeDtypeStruct((NUM_HEADS, SEQ, HEAD_DIM), jnp.float32),
            jax.ShapeDtypeStruct((NUM_HEADS, SEQ, HEAD_DIM), jnp.float32),
            jax.ShapeDtypeStruct((NUM_HEADS, SEQ, HEAD_DIM), jnp.float32),
            jax.ShapeDtypeStruct((NB, 1, HIDDEN), jnp.float32),
        ],
    )(x, w_qkv, cos64, sin64)

    aff = pl.pallas_call(
        _aff_body,
        grid=(1,),
        in_specs=[
            pl.BlockSpec((NUM_HEADS, SEQ, HEAD_DIM), lambda i: (0, 0, 0)),
            pl.BlockSpec((NB, 1, HIDDEN), lambda i: (0, 0, 0)),
        ],
        out_specs=pl.BlockSpec((_NW, _NROW, _CHUNK), lambda i: (0, 0, 0)),
        out_shape=jax.ShapeDtypeStruct((_NW, _NROW, _CHUNK), jnp.float32),
    )(q_all, ksum)

    bias_sc = _get_route()(aff)
    bias = (bias_sc.reshape(_NW, NUM_HEADS, NB, _CHUNK)
            .transpose(1, 0, 3, 2).reshape(NUM_HEADS, SEQ, NB))

    wo_r = Wo.T.reshape(NUM_HEADS, HEAD_DIM, HIDDEN)

    out = pl.pallas_call(
        _attn_body,
        grid=(NUM_HEADS,),
        in_specs=[
            pl.BlockSpec((1, SEQ, HEAD_DIM), lambda h: (h, 0, 0)),
            pl.BlockSpec((1, SEQ, HEAD_DIM), lambda h: (h, 0, 0)),
            pl.BlockSpec((1, SEQ, HEAD_DIM), lambda h: (h, 0, 0)),
            pl.BlockSpec((1, SEQ, NB), lambda h: (h, 0, 0)),
            pl.BlockSpec((NUM_HEADS, HEAD_DIM, HIDDEN), lambda h: (0, 0, 0)),
        ],
        out_specs=pl.BlockSpec((SEQ, HIDDEN), lambda h: (0, 0)),
        out_shape=jax.ShapeDtypeStruct((SEQ, HIDDEN), jnp.float32),
        scratch_shapes=[pltpu.VMEM((NUM_HEADS, SEQ, HEAD_DIM), jnp.float32)],
    )(q_all, k_all, v_all, bias, wo_r)

    return out[None]


# speedup vs baseline: 3.3222x; 1.0286x over previous
import functools

import jax
import jax.numpy as jnp
from jax import lax
from jax.experimental import pallas as pl
from jax.experimental.pallas import tpu as pltpu
from jax.experimental.pallas import tpu_sc as plsc

HIDDEN = 768
NUM_HEADS = 12
HEAD_DIM = 64
BLOCK = 512
SEQ = 2048
NB = SEQ // BLOCK
SCALE = 0.125
NEG = -1e30
LN2 = 0.6931471805599453

_SC_CORES = 2
_SC_SUBCORES = 16
_NW = _SC_CORES * _SC_SUBCORES
_CHUNK = SEQ // _NW
_NROW = NUM_HEADS * NB


def _qkv_body(x_ref, w_ref, cos_ref, sin_ref, q_ref, k_ref, v_ref, aff_ref,
              qbuf_ref, ksum_ref):
    i = pl.program_id(0)

    @pl.when(i < NB)
    def _():
        x = x_ref[...]
        w = w_ref[...]
        y = jnp.dot(x, w, preferred_element_type=jnp.float32)
        cos = cos_ref[...]
        sin = sin_ref[...]
        cos_t = jnp.concatenate([cos] * NUM_HEADS, axis=1)
        sin_t = jnp.concatenate([sin] * NUM_HEADS, axis=1)
        q = y[:, :HIDDEN]
        k = y[:, HIDDEN:2 * HIDDEN]

        def rot(z):
            parts = []
            for h in range(NUM_HEADS):
                zh = z[:, h * HEAD_DIM:(h + 1) * HEAD_DIM]
                parts.append(
                    jnp.concatenate(
                        [-zh[:, HEAD_DIM // 2:], zh[:, :HEAD_DIM // 2]],
                        axis=1))
            return jnp.concatenate(parts, axis=1)

        q = q * cos_t + rot(q) * sin_t
        k = k * cos_t + rot(k) * sin_t
        for h in range(NUM_HEADS):
            qh = q[:, h * HEAD_DIM:(h + 1) * HEAD_DIM]
            q_ref[h] = qh
            qbuf_ref[h, i] = qh
            k_ref[h] = k[:, h * HEAD_DIM:(h + 1) * HEAD_DIM]
            v_ref[h] = y[:, 2 * HIDDEN + h * HEAD_DIM:
                         2 * HIDDEN + (h + 1) * HEAD_DIM]
        ksum_ref[i] = jnp.sum(k, axis=0)[None]

    @pl.when(i == NB)
    def _():
        ks = jnp.concatenate([ksum_ref[n] for n in range(NB)], axis=0)
        _WPB = BLOCK // _CHUNK
        for h in range(NUM_HEADS):
            kh = ks[:, h * HEAD_DIM:(h + 1) * HEAD_DIM]
            for n in range(NB):
                a = lax.dot_general(kh, qbuf_ref[h, n],
                                    (((1,), (1,)), ((), ())),
                                    preferred_element_type=jnp.float32)
                a = a * (SCALE / BLOCK)
                for w in range(_WPB):
                    aff_ref[n * _WPB + w, h * NB:(h + 1) * NB, :] = (
                        a[:, w * _CHUNK:(w + 1) * _CHUNK])


def _route_body(aff_hbm, bias_hbm, abuf, bbuf):
    wid = lax.axis_index("s") * _SC_CORES + lax.axis_index("c")
    base = wid * _CHUNK
    pltpu.sync_copy(aff_hbm.at[wid], abuf)

    def b2i(c):
        return jnp.where(c, jnp.int32(1), jnp.int32(0))

    def load_masked(h, g, cur):
        a = []
        for j in range(NB):
            aj = abuf[h * NB + j, pl.ds(g * 16, 16)]
            a.append(jnp.where(jnp.int32(j) > cur, NEG, aj))
        return a

    def ranks(a):
        r = []
        for j in range(NB):
            rj = None
            for jp in range(NB):
                if jp == j:
                    continue
                c = a[jp] > a[j]
                if jp < j:
                    c = c | (a[jp] == a[j])
                rj = b2i(c) if rj is None else rj + b2i(c)
            r.append(rj)
        return r

    for g in range(_CHUNK // 16):
        ids = lax.broadcasted_iota(jnp.int32, (16,), 0)
        pos = ids + (base + g * 16)
        cur = lax.shift_right_logical(pos, 9)

        allhave = pos >= 0
        for h in range(NUM_HEADS):
            a = load_masked(h, g, cur)
            acur = a[NB - 1]
            for j in range(NB - 2, -1, -1):
                acur = jnp.where(cur == j, a[j], acur)
            rc = None
            for j in range(NB):
                c = (a[j] > acur) | ((a[j] == acur) & (jnp.int32(j) < cur))
                rc = b2i(c) if rc is None else rc + b2i(c)
            allhave = allhave & (rc < 2)

        for h in range(NUM_HEADS):
            a = load_masked(h, g, cur)
            r = ranks(a)
            v0 = jnp.maximum(jnp.maximum(a[0], a[1]), jnp.maximum(a[2], a[3]))
            v1 = jnp.where(r[0] == 1, a[0], NEG)
            for j in range(1, NB):
                v1 = jnp.where(r[j] == 1, a[j], v1)
            tie = v0 == v1
            kept_rank = b2i(tie)
            kept = None
            for j in range(NB):
                kj = jnp.where(r[j] == kept_rank, jnp.int32(j), jnp.int32(0))
                kept = kj if kept is None else kept + kj
            for j in range(NB):
                sel = b2i(r[j] < 2)
                fixed = b2i(kept == j) + b2i(cur == j)
                cnt = jnp.where(allhave, sel, fixed)
                bias = jnp.where(cnt == 0, jnp.float32(NEG),
                                 jnp.where(cnt == 2, jnp.float32(LN2),
                                           jnp.float32(0.0)))
                bbuf[h * NB + j, pl.ds(g * 16, 16)] = bias

    pltpu.sync_copy(bbuf, bias_hbm.at[wid])


@functools.cache
def _get_route():
    return pl.kernel(
        _route_body,
        out_type=jax.ShapeDtypeStruct((_NW, _NROW, _CHUNK), jnp.float32),
        mesh=plsc.VectorSubcoreMesh(
            core_axis_name="c", subcore_axis_name="s",
            num_cores=_SC_CORES, num_subcores=_SC_SUBCORES),
        scratch_types=[
            pltpu.VMEM((_NROW, _CHUNK), jnp.float32),
            pltpu.VMEM((_NROW, _CHUNK), jnp.float32),
        ],
    )


def _attn_body(q_ref, k_ref, v_ref, b_ref, wo_ref, o_ref, acc_ref):
    h = pl.program_id(0)
    q = q_ref[0]
    k = k_ref[0]
    v = v_ref[0]
    b = b_ref[0]
    outs = []
    for qb in range(NB):
        nkb = (2, 2, 3, 4)[qb]
        nk = nkb * BLOCK
        qs = q[qb * BLOCK:(qb + 1) * BLOCK]
        s = lax.dot_general(qs, k[:nk], (((1,), (1,)), ((), ())),
                            preferred_element_type=jnp.float32) * SCALE
        bias = jnp.concatenate(
            [jnp.broadcast_to(b[qb * BLOCK:(qb + 1) * BLOCK, j:j + 1],
                              (BLOCK, BLOCK)) for j in range(nkb)], axis=1)
        s = s + bias
        m = jnp.max(s, axis=1, keepdims=True)
        e = jnp.exp(s - m)
        den = jnp.sum(e, axis=1, keepdims=True)
        o = lax.dot_general(e, v[:nk], (((1,), (0,)), ((), ())),
                            preferred_element_type=jnp.float32) / den
        outs.append(o)
    acc_ref[h] = jnp.concatenate(outs, axis=0)

    @pl.when(h == NUM_HEADS - 1)
    def _():
        acc = None
        for hh in range(NUM_HEADS):
            c = jnp.dot(acc_ref[hh], wo_ref[hh],
                        preferred_element_type=jnp.float32)
            acc = c if acc is None else acc + c
        o_ref[...] = acc


@jax.jit
def kernel(hidden_states, Wq, Wk, Wv, Wo):
    x = hidden_states[0]
    w_qkv = jnp.concatenate([Wq, Wk, Wv], axis=0).T

    inv = 1.0 / (10000.0 ** (jnp.arange(0, HEAD_DIM, 2, dtype=jnp.float32)
                             / HEAD_DIM))
    t = jnp.arange(SEQ, dtype=jnp.float32)
    freqs = t[:, None] * inv[None, :]
    cos64 = jnp.concatenate([jnp.cos(freqs)] * 2, axis=1)
    sin64 = jnp.concatenate([jnp.sin(freqs)] * 2, axis=1)

    _last = NB - 1
    q_all, k_all, v_all, aff = pl.pallas_call(
        _qkv_body,
        grid=(NB + 1,),
        in_specs=[
            pl.BlockSpec((BLOCK, HIDDEN), lambda i: (jnp.minimum(i, _last), 0)),
            pl.BlockSpec((HIDDEN, 3 * HIDDEN), lambda i: (0, 0)),
            pl.BlockSpec((BLOCK, HEAD_DIM), lambda i: (jnp.minimum(i, _last), 0)),
            pl.BlockSpec((BLOCK, HEAD_DIM), lambda i: (jnp.minimum(i, _last), 0)),
        ],
        out_specs=[
            pl.BlockSpec((NUM_HEADS, BLOCK, HEAD_DIM),
                         lambda i: (0, jnp.minimum(i, _last), 0)),
            pl.BlockSpec((NUM_HEADS, BLOCK, HEAD_DIM),
                         lambda i: (0, jnp.minimum(i, _last), 0)),
            pl.BlockSpec((NUM_HEADS, BLOCK, HEAD_DIM),
                         lambda i: (0, jnp.minimum(i, _last), 0)),
            pl.BlockSpec((_NW, _NROW, _CHUNK), lambda i: (0, 0, 0)),
        ],
        out_shape=[
            jax.ShapeDtypeStruct((NUM_HEADS, SEQ, HEAD_DIM), jnp.float32),
            jax.ShapeDtypeStruct((NUM_HEADS, SEQ, HEAD_DIM), jnp.float32),
            jax.ShapeDtypeStruct((NUM_HEADS, SEQ, HEAD_DIM), jnp.float32),
            jax.ShapeDtypeStruct((_NW, _NROW, _CHUNK), jnp.float32),
        ],
        scratch_shapes=[
            pltpu.VMEM((NUM_HEADS, NB, BLOCK, HEAD_DIM), jnp.float32),
            pltpu.VMEM((NB, 1, HIDDEN), jnp.float32),
        ],
    )(x, w_qkv, cos64, sin64)

    bias_sc = _get_route()(aff)
    bias = (bias_sc.reshape(_NW, NUM_HEADS, NB, _CHUNK)
            .transpose(1, 0, 3, 2).reshape(NUM_HEADS, SEQ, NB))

    wo_r = Wo.T.reshape(NUM_HEADS, HEAD_DIM, HIDDEN)

    out = pl.pallas_call(
        _attn_body,
        grid=(NUM_HEADS,),
        in_specs=[
            pl.BlockSpec((1, SEQ, HEAD_DIM), lambda h: (h, 0, 0)),
            pl.BlockSpec((1, SEQ, HEAD_DIM), lambda h: (h, 0, 0)),
            pl.BlockSpec((1, SEQ, HEAD_DIM), lambda h: (h, 0, 0)),
            pl.BlockSpec((1, SEQ, NB), lambda h: (h, 0, 0)),
            pl.BlockSpec((NUM_HEADS, HEAD_DIM, HIDDEN), lambda h: (0, 0, 0)),
        ],
        out_specs=pl.BlockSpec((SEQ, HIDDEN), lambda h: (0, 0)),
        out_shape=jax.ShapeDtypeStruct((SEQ, HIDDEN), jnp.float32),
        scratch_shapes=[pltpu.VMEM((NUM_HEADS, SEQ, HEAD_DIM), jnp.float32)],
    )(q_all, k_all, v_all, bias, wo_r)

    return out[None]


# speedup vs baseline: 3.6354x; 1.0943x over previous
import functools

import jax
import jax.numpy as jnp
from jax import lax
from jax.experimental import pallas as pl
from jax.experimental.pallas import tpu as pltpu
from jax.experimental.pallas import tpu_sc as plsc

HIDDEN = 768
NUM_HEADS = 12
HEAD_DIM = 64
BLOCK = 512
SEQ = 2048
NB = SEQ // BLOCK
SCALE = 0.125
NEG = -1e30
LN2 = 0.6931471805599453

_SC_CORES = 2
_SC_SUBCORES = 16
_NW = _SC_CORES * _SC_SUBCORES
_CHUNK = SEQ // _NW
_NROW = NUM_HEADS * NB


def _qkv_body(x_ref, wq_ref, wk_ref, wv_ref, cos_ref, sin_ref,
              q_ref, k_ref, v_ref, aff_ref, qbuf_ref, ksum_ref):
    i = pl.program_id(0)

    @pl.when(i < NB)
    def _():
        x = x_ref[...]
        cn = (((1,), (1,)), ((), ()))
        q = lax.dot_general(x, wq_ref[...], cn,
                            preferred_element_type=jnp.float32)
        k = lax.dot_general(x, wk_ref[...], cn,
                            preferred_element_type=jnp.float32)
        v = lax.dot_general(x, wv_ref[...], cn,
                            preferred_element_type=jnp.float32)
        cos = cos_ref[...]
        sin = sin_ref[...]
        cos_t = jnp.concatenate([cos] * NUM_HEADS, axis=1)
        sin_t = jnp.concatenate([sin] * NUM_HEADS, axis=1)

        def rot(z):
            parts = []
            for h in range(NUM_HEADS):
                zh = z[:, h * HEAD_DIM:(h + 1) * HEAD_DIM]
                parts.append(
                    jnp.concatenate(
                        [-zh[:, HEAD_DIM // 2:], zh[:, :HEAD_DIM // 2]],
                        axis=1))
            return jnp.concatenate(parts, axis=1)

        q = q * cos_t + rot(q) * sin_t
        k = k * cos_t + rot(k) * sin_t
        for h in range(NUM_HEADS):
            qh = q[:, h * HEAD_DIM:(h + 1) * HEAD_DIM]
            q_ref[h] = qh
            qbuf_ref[h, i] = qh
            k_ref[h] = k[:, h * HEAD_DIM:(h + 1) * HEAD_DIM]
            v_ref[h] = v[:, h * HEAD_DIM:(h + 1) * HEAD_DIM]
        ksum_ref[i] = jnp.sum(k, axis=0)[None]

    @pl.when(i == NB)
    def _():
        ks = jnp.concatenate([ksum_ref[n] for n in range(NB)], axis=0)
        _WPB = BLOCK // _CHUNK
        for h in range(NUM_HEADS):
            kh = ks[:, h * HEAD_DIM:(h + 1) * HEAD_DIM]
            for n in range(NB):
                a = lax.dot_general(kh, qbuf_ref[h, n],
                                    (((1,), (1,)), ((), ())),
                                    preferred_element_type=jnp.float32)
                a = a * (SCALE / BLOCK)
                for w in range(_WPB):
                    aff_ref[n * _WPB + w, h * NB:(h + 1) * NB, :] = (
                        a[:, w * _CHUNK:(w + 1) * _CHUNK])


def _route_body(aff_hbm, bias_hbm, abuf, bbuf):
    wid = lax.axis_index("s") * _SC_CORES + lax.axis_index("c")
    base = wid * _CHUNK
    pltpu.sync_copy(aff_hbm.at[wid], abuf)

    def b2i(c):
        return jnp.where(c, jnp.int32(1), jnp.int32(0))

    def load_masked(h, g, cur):
        a = []
        for j in range(NB):
            aj = abuf[h * NB + j, pl.ds(g * 16, 16)]
            a.append(jnp.where(jnp.int32(j) > cur, NEG, aj))
        return a

    def ranks(a):
        r = []
        for j in range(NB):
            rj = None
            for jp in range(NB):
                if jp == j:
                    continue
                c = a[jp] > a[j]
                if jp < j:
                    c = c | (a[jp] == a[j])
                rj = b2i(c) if rj is None else rj + b2i(c)
            r.append(rj)
        return r

    for g in range(_CHUNK // 16):
        ids = lax.broadcasted_iota(jnp.int32, (16,), 0)
        pos = ids + (base + g * 16)
        cur = lax.shift_right_logical(pos, 9)

        allhave = pos >= 0
        for h in range(NUM_HEADS):
            a = load_masked(h, g, cur)
            acur = a[NB - 1]
            for j in range(NB - 2, -1, -1):
                acur = jnp.where(cur == j, a[j], acur)
            rc = None
            for j in range(NB):
                c = (a[j] > acur) | ((a[j] == acur) & (jnp.int32(j) < cur))
                rc = b2i(c) if rc is None else rc + b2i(c)
            allhave = allhave & (rc < 2)

        for h in range(NUM_HEADS):
            a = load_masked(h, g, cur)
            r = ranks(a)
            v0 = jnp.maximum(jnp.maximum(a[0], a[1]), jnp.maximum(a[2], a[3]))
            v1 = jnp.where(r[0] == 1, a[0], NEG)
            for j in range(1, NB):
                v1 = jnp.where(r[j] == 1, a[j], v1)
            tie = v0 == v1
            kept_rank = b2i(tie)
            kept = None
            for j in range(NB):
                kj = jnp.where(r[j] == kept_rank, jnp.int32(j), jnp.int32(0))
                kept = kj if kept is None else kept + kj
            for j in range(NB):
                sel = b2i(r[j] < 2)
                fixed = b2i(kept == j) + b2i(cur == j)
                cnt = jnp.where(allhave, sel, fixed)
                bias = jnp.where(cnt == 0, jnp.float32(NEG),
                                 jnp.where(cnt == 2, jnp.float32(LN2),
                                           jnp.float32(0.0)))
                bbuf[h * NB + j, pl.ds(g * 16, 16)] = bias

    pltpu.sync_copy(bbuf, bias_hbm.at[wid])


@functools.cache
def _get_route():
    return pl.kernel(
        _route_body,
        out_type=jax.ShapeDtypeStruct((_NW, _NROW, _CHUNK), jnp.float32),
        mesh=plsc.VectorSubcoreMesh(
            core_axis_name="c", subcore_axis_name="s",
            num_cores=_SC_CORES, num_subcores=_SC_SUBCORES),
        scratch_types=[
            pltpu.VMEM((_NROW, _CHUNK), jnp.float32),
            pltpu.VMEM((_NROW, _CHUNK), jnp.float32),
        ],
    )


def _attn_body(q_ref, k_ref, v_ref, b_ref, wo_ref, o_ref, acc_ref):
    h = pl.program_id(0)
    q = q_ref[0]
    k = k_ref[0]
    v = v_ref[0]
    b = b_ref[0]
    outs = []
    for qb in range(NB):
        nkb = (2, 2, 3, 4)[qb]
        nk = nkb * BLOCK
        qs = q[qb * BLOCK:(qb + 1) * BLOCK]
        s = lax.dot_general(qs, k[:nk], (((1,), (1,)), ((), ())),
                            preferred_element_type=jnp.float32) * SCALE
        bias = jnp.concatenate(
            [jnp.broadcast_to(b[qb * BLOCK:(qb + 1) * BLOCK, j:j + 1],
                              (BLOCK, BLOCK)) for j in range(nkb)], axis=1)
        s = s + bias
        m = jnp.max(s, axis=1, keepdims=True)
        e = jnp.exp(s - m)
        den = jnp.sum(e, axis=1, keepdims=True)
        o = lax.dot_general(e, v[:nk], (((1,), (0,)), ((), ())),
                            preferred_element_type=jnp.float32) / den
        outs.append(o)
    acc_ref[h] = jnp.concatenate(outs, axis=0)

    @pl.when(h == NUM_HEADS - 1)
    def _():
        acc = None
        for hh in range(NUM_HEADS):
            c = lax.dot_general(
                acc_ref[hh], wo_ref[:, hh * HEAD_DIM:(hh + 1) * HEAD_DIM],
                (((1,), (1,)), ((), ())), preferred_element_type=jnp.float32)
            acc = c if acc is None else acc + c
        o_ref[...] = acc


@jax.jit
def kernel(hidden_states, Wq, Wk, Wv, Wo):
    x = hidden_states[0]

    inv = 1.0 / (10000.0 ** (jnp.arange(0, HEAD_DIM, 2, dtype=jnp.float32)
                             / HEAD_DIM))
    t = jnp.arange(SEQ, dtype=jnp.float32)
    freqs = t[:, None] * inv[None, :]
    cos64 = jnp.concatenate([jnp.cos(freqs)] * 2, axis=1)
    sin64 = jnp.concatenate([jnp.sin(freqs)] * 2, axis=1)

    _last = NB - 1
    q_all, k_all, v_all, aff = pl.pallas_call(
        _qkv_body,
        grid=(NB + 1,),
        in_specs=[
            pl.BlockSpec((BLOCK, HIDDEN), lambda i: (jnp.minimum(i, _last), 0)),
            pl.BlockSpec((HIDDEN, HIDDEN), lambda i: (0, 0)),
            pl.BlockSpec((HIDDEN, HIDDEN), lambda i: (0, 0)),
            pl.BlockSpec((HIDDEN, HIDDEN), lambda i: (0, 0)),
            pl.BlockSpec((BLOCK, HEAD_DIM), lambda i: (jnp.minimum(i, _last), 0)),
            pl.BlockSpec((BLOCK, HEAD_DIM), lambda i: (jnp.minimum(i, _last), 0)),
        ],
        out_specs=[
            pl.BlockSpec((NUM_HEADS, BLOCK, HEAD_DIM),
                         lambda i: (0, jnp.minimum(i, _last), 0)),
            pl.BlockSpec((NUM_HEADS, BLOCK, HEAD_DIM),
                         lambda i: (0, jnp.minimum(i, _last), 0)),
            pl.BlockSpec((NUM_HEADS, BLOCK, HEAD_DIM),
                         lambda i: (0, jnp.minimum(i, _last), 0)),
            pl.BlockSpec((_NW, _NROW, _CHUNK), lambda i: (0, 0, 0)),
        ],
        out_shape=[
            jax.ShapeDtypeStruct((NUM_HEADS, SEQ, HEAD_DIM), jnp.float32),
            jax.ShapeDtypeStruct((NUM_HEADS, SEQ, HEAD_DIM), jnp.float32),
            jax.ShapeDtypeStruct((NUM_HEADS, SEQ, HEAD_DIM), jnp.float32),
            jax.ShapeDtypeStruct((_NW, _NROW, _CHUNK), jnp.float32),
        ],
        scratch_shapes=[
            pltpu.VMEM((NUM_HEADS, NB, BLOCK, HEAD_DIM), jnp.float32),
            pltpu.VMEM((NB, 1, HIDDEN), jnp.float32),
        ],
    )(x, Wq, Wk, Wv, cos64, sin64)

    bias_sc = _get_route()(aff)
    bias = (bias_sc.reshape(_NW, NUM_HEADS, NB, _CHUNK)
            .transpose(1, 0, 3, 2).reshape(NUM_HEADS, SEQ, NB))

    out = pl.pallas_call(
        _attn_body,
        grid=(NUM_HEADS,),
        in_specs=[
            pl.BlockSpec((1, SEQ, HEAD_DIM), lambda h: (h, 0, 0)),
            pl.BlockSpec((1, SEQ, HEAD_DIM), lambda h: (h, 0, 0)),
            pl.BlockSpec((1, SEQ, HEAD_DIM), lambda h: (h, 0, 0)),
            pl.BlockSpec((1, SEQ, NB), lambda h: (h, 0, 0)),
            pl.BlockSpec((HIDDEN, HIDDEN), lambda h: (0, 0)),
        ],
        out_specs=pl.BlockSpec((SEQ, HIDDEN), lambda h: (0, 0)),
        out_shape=jax.ShapeDtypeStruct((SEQ, HIDDEN), jnp.float32),
        scratch_shapes=[pltpu.VMEM((NUM_HEADS, SEQ, HEAD_DIM), jnp.float32)],
    )(q_all, k_all, v_all, bias, Wo)

    return out[None]
